# baseline (speedup 1.0000x reference)
import jax
import jax.numpy as jnp
from jax.experimental import pallas as pl
from jax.experimental.pallas import tpu as pltpu

_LEVEL_HW = [(100, 152), (50, 76), (25, 38), (13, 19), (7, 10)]
_STRIDES = [8, 16, 32, 64, 128]
_LEVEL_SIZES = [h * w for h, w in _LEVEL_HW]
_N = sum(_LEVEL_SIZES)
_NP = 160 * 128
_NCHUNK = 160
_NUM_CLASS = 80
_K = 1000
_SCORE_THR = 0.05
_IOU_THR = 0.5
_MAX_DET = 100
_IMG_H, _IMG_W = 800.0, 1216.0
_NEG = -1e9
_IMIN = -2**31


def _reduce_kernel(clsT_ref, ct_ref, ms_ref, fsc_ref, lab_ref):
    sig = jax.nn.sigmoid(clsT_ref[...])
    ctv = jax.nn.sigmoid(ct_ref[0])
    ms_ref[0] = jnp.max(sig, axis=0, keepdims=True)
    fin = sig * ctv
    fsc = jnp.max(fin, axis=0, keepdims=True)
    fsc_ref[0] = fsc
    ci = jax.lax.broadcasted_iota(jnp.int32, (128, 128), 0).astype(jnp.float32)
    lab_ref[0] = jnp.min(jnp.where(fin == fsc, ci, 1e9), axis=0, keepdims=True)


def _topk2(skey0, gi0, mask0, skey1, gi1, mask1, k):
    def val_bit(i, c):
        us0, us1 = c
        b = 29 - i
        one = jnp.left_shift(jnp.int32(1), b)
        cand0, cand1 = us0 | one, us1 | one
        n0 = jnp.sum((mask0 & (skey0 >= (cand0 ^ _IMIN))).astype(jnp.int32))
        n1 = jnp.sum((mask1 & (skey1 >= (cand1 ^ _IMIN))).astype(jnp.int32))
        return (jnp.where(n0 >= k, cand0, us0), jnp.where(n1 >= k, cand1, us1))
    us0, us1 = jax.lax.fori_loop(0, 30, val_bit, (jnp.int32(_IMIN), jnp.int32(_IMIN)))
    v0, v1 = us0 ^ _IMIN, us1 ^ _IMIN
    cgt0 = jnp.sum((mask0 & (skey0 > v0)).astype(jnp.int32))
    cgt1 = jnp.sum((mask1 & (skey1 > v1)).astype(jnp.int32))
    need0, need1 = k - cgt0, k - cgt1
    ties0 = mask0 & (skey0 == v0)
    ties1 = mask1 & (skey1 == v1)
    def idx_bit(i, c):
        t0, t1 = c
        b = 14 - i
        one = jnp.left_shift(jnp.int32(1), b)
        cand0, cand1 = t0 | one, t1 | one
        n0 = jnp.sum((ties0 & (gi0 < cand0)).astype(jnp.int32))
        n1 = jnp.sum((ties1 & (gi1 < cand1)).astype(jnp.int32))
        return (jnp.where(n0 <= need0, cand0, t0), jnp.where(n1 <= need1, cand1, t1))
    ti0, ti1 = jax.lax.fori_loop(0, 15, idx_bit, (jnp.int32(0), jnp.int32(0)))
    return v0, ti0, v1, ti1


def _nms_kernel(ms_ref, fsc_ref, lab_ref, d0_ref, d1_ref, d2_ref, d3_ref,
                mx_ref, my_ref, out_ref):
    ms = ms_ref[...]
    fsc = fsc_ref[...]
    lab = lab_ref[...]
    mx = mx_ref[...]
    my = my_ref[...]
    r_iota = jax.lax.broadcasted_iota(jnp.int32, (_NCHUNK, 128), 0)
    c_iota = jax.lax.broadcasted_iota(jnp.int32, (_NCHUNK, 128), 1)
    gi = r_iota * 128 + c_iota
    valid = gi < _N

    x1 = jnp.clip(mx - d0_ref[...], 0.0, _IMG_W)
    y1 = jnp.clip(my - d1_ref[...], 0.0, _IMG_H)
    x2 = jnp.clip(mx + d2_ref[...], 0.0, _IMG_W)
    y2 = jnp.clip(my + d3_ref[...], 0.0, _IMG_H)

    u = jax.lax.bitcast_convert_type(ms, jnp.int32)
    skey = jnp.where(u >= 0, u, u ^ jnp.int32(0x7FFFFFFF))
    skey = jnp.where(valid, skey, _IMIN)

    n0, n1 = _LEVEL_SIZES[0], _LEVEL_SIZES[0] + _LEVEL_SIZES[1]
    r0 = (n0 + 127) // 128
    r1lo, r1hi = n0 // 128, (n1 + 127) // 128
    v0, ti0, v1, ti1 = _topk2(
        skey[:r0], gi[:r0], gi[:r0] < n0,
        skey[r1lo:r1hi], gi[r1lo:r1hi],
        (gi[r1lo:r1hi] >= n0) & (gi[r1lo:r1hi] < n1), _K)
    sel0 = (gi < n0) & ((skey > v0) | ((skey == v0) & (gi < ti0)))
    sel1 = (gi >= n0) & (gi < n1) & ((skey > v1) | ((skey == v1) & (gi < ti1)))
    selected = (valid & (gi >= n1)) | sel0 | sel1

    bounds = [sum(_LEVEL_SIZES[:i + 1]) for i in range(5)]
    level = jnp.zeros((_NCHUNK, 128), jnp.int32)
    for li in range(1, 5):
        level = jnp.where(gi >= bounds[li - 1], li, level)

    fs = jnp.where(fsc > _SCORE_THR, fsc, _NEG)
    fs = jnp.where(selected, fs, _NEG)

    ci_row = jax.lax.broadcasted_iota(jnp.int32, (1, 128), 1)
    big = jnp.int32(2**30)
    a2 = (x2 - x1) * (y2 - y1)

    ax = (0, 1)

    def body(i, fs):
        bsc = jnp.max(fs, axis=ax, keepdims=True)
        t1 = fs == bsc
        blev = jnp.min(jnp.where(t1, level, big), axis=ax, keepdims=True)
        t2 = t1 & (level == blev)
        bms = jnp.max(jnp.where(t2, ms, _NEG), axis=ax, keepdims=True)
        t3 = t2 & (ms == bms)
        bi = jnp.min(jnp.where(t3, gi, big), axis=ax, keepdims=True)
        pick = gi == bi
        pf = pick.astype(jnp.float32)
        bx1 = jnp.sum(pf * x1, axis=ax, keepdims=True)
        by1 = jnp.sum(pf * y1, axis=ax, keepdims=True)
        bx2 = jnp.sum(pf * x2, axis=ax, keepdims=True)
        by2 = jnp.sum(pf * y2, axis=ax, keepdims=True)
        blab = jnp.sum(pf * lab, axis=ax, keepdims=True)
        ok = bsc > 0.0
        row = (jnp.where(ci_row == 0, bx1, 0.0) + jnp.where(ci_row == 1, by1, 0.0)
               + jnp.where(ci_row == 2, bx2, 0.0) + jnp.where(ci_row == 3, by2, 0.0)
               + jnp.where(ci_row == 4, bsc, 0.0) + jnp.where(ci_row == 5, blab, 0.0))
        out_ref[pl.ds(i, 1), :] = jnp.where(ok, row, 0.0)
        ix1 = jnp.maximum(bx1, x1)
        iy1 = jnp.maximum(by1, y1)
        ix2 = jnp.minimum(bx2, x2)
        iy2 = jnp.minimum(by2, y2)
        inter = jnp.maximum(ix2 - ix1, 0.0) * jnp.maximum(iy2 - iy1, 0.0)
        a1 = (bx2 - bx1) * (by2 - by1)
        iou = inter / (a1 + a2 - inter + 1e-6)
        supp = (iou > _IOU_THR) & (lab == blab)
        return jnp.where(supp | pick, _NEG, fs)

    jax.lax.fori_loop(0, _MAX_DET, body, fs)


def kernel(pred_class, pred_bbox, pred_centerness, mesh):
    padn = _NP - _N
    clsT = jnp.pad(pred_class, ((0, padn), (0, 128 - _NUM_CLASS)),
                   constant_values=_NEG).T
    ct = jnp.pad(pred_centerness, (0, padn)).reshape(_NCHUNK, 1, 128)
    pb = jnp.pad(pred_bbox, ((0, padn), (0, 0)))
    d0 = pb[:, 0].reshape(_NCHUNK, 128)
    d1 = pb[:, 1].reshape(_NCHUNK, 128)
    d2 = pb[:, 2].reshape(_NCHUNK, 128)
    d3 = pb[:, 3].reshape(_NCHUNK, 128)
    mp = jnp.pad(mesh, ((0, padn), (0, 0)))
    mx = mp[:, 0].reshape(_NCHUNK, 128)
    my = mp[:, 1].reshape(_NCHUNK, 128)

    f32 = jnp.float32
    ms, fsc, lab = pl.pallas_call(
        _reduce_kernel,
        grid=(_NCHUNK,),
        in_specs=[
            pl.BlockSpec((128, 128), lambda i: (0, i)),
            pl.BlockSpec((1, 1, 128), lambda i: (i, 0, 0)),
        ],
        out_specs=[
            pl.BlockSpec((1, 1, 128), lambda i: (i, 0, 0)),
            pl.BlockSpec((1, 1, 128), lambda i: (i, 0, 0)),
            pl.BlockSpec((1, 1, 128), lambda i: (i, 0, 0)),
        ],
        out_shape=[
            jax.ShapeDtypeStruct((_NCHUNK, 1, 128), f32),
            jax.ShapeDtypeStruct((_NCHUNK, 1, 128), f32),
            jax.ShapeDtypeStruct((_NCHUNK, 1, 128), f32),
        ],
    )(clsT, ct)
    ms = ms.reshape(_NCHUNK, 128)
    fsc = fsc.reshape(_NCHUNK, 128)
    lab = lab.reshape(_NCHUNK, 128)

    out = pl.pallas_call(
        _nms_kernel,
        out_shape=jax.ShapeDtypeStruct((104, 128), f32),
    )(ms, fsc, lab, d0, d1, d2, d3, mx, my)
    return out[:_MAX_DET, :6]


# speedup vs baseline: 1.2420x; 1.2420x over previous
import jax
import jax.numpy as jnp
from jax.experimental import pallas as pl
from jax.experimental.pallas import tpu as pltpu

_LEVEL_HW = [(100, 152), (50, 76), (25, 38), (13, 19), (7, 10)]
_STRIDES = [8, 16, 32, 64, 128]
_LEVEL_SIZES = [h * w for h, w in _LEVEL_HW]
_N = sum(_LEVEL_SIZES)
_NP = 160 * 128
_NCHUNK = 160
_NUM_CLASS = 80
_K = 1000
_SCORE_THR = 0.05
_IOU_THR = 0.5
_MAX_DET = 100
_IMG_H, _IMG_W = 800.0, 1216.0
_NEG = -1e9
_IMIN = -2**31


def _reduce_kernel(clsT_ref, ct_ref, ms_ref, fsc_ref, lab_ref):
    sig = jax.nn.sigmoid(clsT_ref[...])
    ctv = jax.nn.sigmoid(ct_ref[0])
    ms_ref[0] = jnp.max(sig, axis=0, keepdims=True)
    fin = sig * ctv
    fsc = jnp.max(fin, axis=0, keepdims=True)
    fsc_ref[0] = fsc
    ci = jax.lax.broadcasted_iota(jnp.int32, (128, 128), 0).astype(jnp.float32)
    lab_ref[0] = jnp.min(jnp.where(fin == fsc, ci, 1e9), axis=0, keepdims=True)


def _topk2(skey0, gi0, mask0, skey1, gi1, mask1, k):
    def val_bit(i, c):
        us0, us1 = c
        b = 29 - i
        one = jnp.left_shift(jnp.int32(1), b)
        cand0, cand1 = us0 | one, us1 | one
        n0 = jnp.sum((mask0 & (skey0 >= (cand0 ^ _IMIN))).astype(jnp.int32))
        n1 = jnp.sum((mask1 & (skey1 >= (cand1 ^ _IMIN))).astype(jnp.int32))
        return (jnp.where(n0 >= k, cand0, us0), jnp.where(n1 >= k, cand1, us1))
    us0, us1 = jax.lax.fori_loop(0, 30, val_bit, (jnp.int32(_IMIN), jnp.int32(_IMIN)))
    v0, v1 = us0 ^ _IMIN, us1 ^ _IMIN
    cgt0 = jnp.sum((mask0 & (skey0 > v0)).astype(jnp.int32))
    cgt1 = jnp.sum((mask1 & (skey1 > v1)).astype(jnp.int32))
    need0, need1 = k - cgt0, k - cgt1
    ties0 = mask0 & (skey0 == v0)
    ties1 = mask1 & (skey1 == v1)
    def idx_bit(i, c):
        t0, t1 = c
        b = 14 - i
        one = jnp.left_shift(jnp.int32(1), b)
        cand0, cand1 = t0 | one, t1 | one
        n0 = jnp.sum((ties0 & (gi0 < cand0)).astype(jnp.int32))
        n1 = jnp.sum((ties1 & (gi1 < cand1)).astype(jnp.int32))
        return (jnp.where(n0 <= need0, cand0, t0), jnp.where(n1 <= need1, cand1, t1))
    ti0, ti1 = jax.lax.fori_loop(0, 15, idx_bit, (jnp.int32(0), jnp.int32(0)))
    return v0, ti0, v1, ti1


def _nms_kernel(ms_ref, fsc_ref, lab_ref, d0_ref, d1_ref, d2_ref, d3_ref,
                mx_ref, my_ref, out_ref, x1_ref, y1_ref, x2_ref, y2_ref):
    ms = ms_ref[...]
    fsc = fsc_ref[...]
    lab = lab_ref[...]
    mx = mx_ref[...]
    my = my_ref[...]
    r_iota = jax.lax.broadcasted_iota(jnp.int32, (_NCHUNK, 128), 0)
    c_iota = jax.lax.broadcasted_iota(jnp.int32, (_NCHUNK, 128), 1)
    gi = r_iota * 128 + c_iota
    valid = gi < _N

    x1 = jnp.clip(mx - d0_ref[...], 0.0, _IMG_W)
    y1 = jnp.clip(my - d1_ref[...], 0.0, _IMG_H)
    x2 = jnp.clip(mx + d2_ref[...], 0.0, _IMG_W)
    y2 = jnp.clip(my + d3_ref[...], 0.0, _IMG_H)
    x1_ref[...] = x1
    y1_ref[...] = y1
    x2_ref[...] = x2
    y2_ref[...] = y2

    u = jax.lax.bitcast_convert_type(ms, jnp.int32)
    skey = jnp.where(u >= 0, u, u ^ jnp.int32(0x7FFFFFFF))
    skey = jnp.where(valid, skey, _IMIN)

    n0, n1 = _LEVEL_SIZES[0], _LEVEL_SIZES[0] + _LEVEL_SIZES[1]
    r0 = (n0 + 127) // 128
    r1lo, r1hi = n0 // 128, (n1 + 127) // 128
    v0, ti0, v1, ti1 = _topk2(
        skey[:r0], gi[:r0], gi[:r0] < n0,
        skey[r1lo:r1hi], gi[r1lo:r1hi],
        (gi[r1lo:r1hi] >= n0) & (gi[r1lo:r1hi] < n1), _K)
    sel0 = (gi < n0) & ((skey > v0) | ((skey == v0) & (gi < ti0)))
    sel1 = (gi >= n0) & (gi < n1) & ((skey > v1) | ((skey == v1) & (gi < ti1)))
    selected = (valid & (gi >= n1)) | sel0 | sel1

    bounds = [sum(_LEVEL_SIZES[:i + 1]) for i in range(5)]
    level = jnp.zeros((_NCHUNK, 128), jnp.int32)
    for li in range(1, 5):
        level = jnp.where(gi >= bounds[li - 1], li, level)

    fs = jnp.where(fsc > _SCORE_THR, fsc, _NEG)
    fs = jnp.where(selected, fs, _NEG)

    ci_row = jax.lax.broadcasted_iota(jnp.int32, (1, 128), 1)
    big = jnp.int32(2**30)
    a2 = (x2 - x1) * (y2 - y1)

    def body(i, fs):
        bsc = jnp.max(fs)
        t1 = fs == bsc
        cnt = jnp.sum(t1.astype(jnp.int32))
        bi0 = jnp.min(jnp.where(t1, gi, big))

        def tie_break():
            blev = jnp.min(jnp.where(t1, level, big))
            t2 = t1 & (level == blev)
            bms = jnp.max(jnp.where(t2, ms, _NEG))
            t3 = t2 & (ms == bms)
            return jnp.min(jnp.where(t3, gi, big))

        bi = jax.lax.cond(cnt > 1, tie_break, lambda: bi0)
        r = bi >> 7
        c = bi & 127
        onehot = (ci_row == c).astype(jnp.float32)
        bx1 = jnp.sum(x1_ref[pl.ds(r, 1), :] * onehot)
        by1 = jnp.sum(y1_ref[pl.ds(r, 1), :] * onehot)
        bx2 = jnp.sum(x2_ref[pl.ds(r, 1), :] * onehot)
        by2 = jnp.sum(y2_ref[pl.ds(r, 1), :] * onehot)
        blab = jnp.sum(lab_ref[pl.ds(r, 1), :] * onehot)
        ok = bsc > 0.0
        row = (jnp.where(ci_row == 0, bx1, 0.0) + jnp.where(ci_row == 1, by1, 0.0)
               + jnp.where(ci_row == 2, bx2, 0.0) + jnp.where(ci_row == 3, by2, 0.0)
               + jnp.where(ci_row == 4, bsc, 0.0) + jnp.where(ci_row == 5, blab, 0.0))
        out_ref[pl.ds(i, 1), :] = jnp.where(ok, row, 0.0)
        ix1 = jnp.maximum(bx1, x1)
        iy1 = jnp.maximum(by1, y1)
        ix2 = jnp.minimum(bx2, x2)
        iy2 = jnp.minimum(by2, y2)
        inter = jnp.maximum(ix2 - ix1, 0.0) * jnp.maximum(iy2 - iy1, 0.0)
        a1 = (bx2 - bx1) * (by2 - by1)
        iou = inter / (a1 + a2 - inter + 1e-6)
        supp = (iou > _IOU_THR) & (lab == blab)
        pick = gi == bi
        return jnp.where(supp | pick, _NEG, fs)

    jax.lax.fori_loop(0, _MAX_DET, body, fs)


def kernel(pred_class, pred_bbox, pred_centerness, mesh):
    padn = _NP - _N
    clsT = jnp.pad(pred_class, ((0, padn), (0, 128 - _NUM_CLASS)),
                   constant_values=_NEG).T
    ct = jnp.pad(pred_centerness, (0, padn)).reshape(_NCHUNK, 1, 128)
    pb = jnp.pad(pred_bbox, ((0, padn), (0, 0)))
    d0 = pb[:, 0].reshape(_NCHUNK, 128)
    d1 = pb[:, 1].reshape(_NCHUNK, 128)
    d2 = pb[:, 2].reshape(_NCHUNK, 128)
    d3 = pb[:, 3].reshape(_NCHUNK, 128)
    mp = jnp.pad(mesh, ((0, padn), (0, 0)))
    mx = mp[:, 0].reshape(_NCHUNK, 128)
    my = mp[:, 1].reshape(_NCHUNK, 128)

    f32 = jnp.float32
    ms, fsc, lab = pl.pallas_call(
        _reduce_kernel,
        grid=(_NCHUNK,),
        in_specs=[
            pl.BlockSpec((128, 128), lambda i: (0, i)),
            pl.BlockSpec((1, 1, 128), lambda i: (i, 0, 0)),
        ],
        out_specs=[
            pl.BlockSpec((1, 1, 128), lambda i: (i, 0, 0)),
            pl.BlockSpec((1, 1, 128), lambda i: (i, 0, 0)),
            pl.BlockSpec((1, 1, 128), lambda i: (i, 0, 0)),
        ],
        out_shape=[
            jax.ShapeDtypeStruct((_NCHUNK, 1, 128), f32),
            jax.ShapeDtypeStruct((_NCHUNK, 1, 128), f32),
            jax.ShapeDtypeStruct((_NCHUNK, 1, 128), f32),
        ],
    )(clsT, ct)
    ms = ms.reshape(_NCHUNK, 128)
    fsc = fsc.reshape(_NCHUNK, 128)
    lab = lab.reshape(_NCHUNK, 128)

    out = pl.pallas_call(
        _nms_kernel,
        out_shape=jax.ShapeDtypeStruct((104, 128), f32),
        scratch_shapes=[pltpu.VMEM((_NCHUNK, 128), f32) for _ in range(4)],
    )(ms, fsc, lab, d0, d1, d2, d3, mx, my)
    return out[:_MAX_DET, :6]
